# baseline (speedup 1.0000x reference)
import functools

import jax
import jax.numpy as jnp
from jax import lax
from jax.experimental import pallas as pl
from jax.experimental.pallas import tpu as pltpu
from jax.experimental.pallas import tpu_sc as plsc

NC = 2
NS = 16
L = 16
CB = 128


def _project_body(w_ref, e_ref, o0_ref, o1_ref):
    r = lax.dot_general(
        w_ref[...], e_ref[...], (((1,), (0,)), ((), ())),
        preferred_element_type=jnp.float32,
        precision=lax.Precision.HIGHEST)
    o0_ref[...] = r[0]
    o1_ref[...] = r[1]


def _project_table(Wp8, et):
    V = et.shape[1]
    C = 50176
    grid = pl.cdiv(V, C)
    return pl.pallas_call(
        _project_body,
        grid=(grid,),
        in_specs=[
            pl.BlockSpec((8, et.shape[0]), lambda i: (0, 0)),
            pl.BlockSpec((et.shape[0], C), lambda i: (0, i)),
        ],
        out_specs=[
            pl.BlockSpec((C,), lambda i: (i,)),
            pl.BlockSpec((C,), lambda i: (i,)),
        ],
        out_shape=[
            jax.ShapeDtypeStruct((V,), jnp.float32),
            jax.ShapeDtypeStruct((V,), jnp.float32),
        ],
    )(Wp8, et)


def _epilogue_body(a_ref, bias_ref, o_ref):
    z = a_ref[...] + bias_ref[...]
    s = jax.nn.sigmoid(z)
    m = jnp.max(s, axis=0, keepdims=True)
    lse = m + jnp.log(jnp.sum(jnp.exp(s - m), axis=0, keepdims=True))
    o_ref[...] = s - lse


def _epilogue(acc2, bias_col):
    B = acc2.shape[1]
    BLK = 4096
    return pl.pallas_call(
        _epilogue_body,
        grid=(B // BLK,),
        in_specs=[
            pl.BlockSpec((2, BLK), lambda i: (0, i)),
            pl.BlockSpec((2, 1), lambda i: (0, 0)),
        ],
        out_specs=pl.BlockSpec((2, BLK), lambda i: (0, i)),
        out_shape=jax.ShapeDtypeStruct((2, B), jnp.float32),
    )(acc2, bias_col)


def _make_sc_sum(S, B):
    NW = NC * NS
    BPW = B // NW
    NCHUNK = BPW // CB
    mesh = plsc.VectorSubcoreMesh(
        core_axis_name="c", subcore_axis_name="s",
        num_cores=NC, num_subcores=NS)

    NB = 8

    @functools.partial(
        pl.kernel,
        out_type=jax.ShapeDtypeStruct((2, B), jnp.float32),
        mesh=mesh,
        compiler_params=pltpu.CompilerParams(use_tc_tiling_on_sc=False),
        scratch_types=[
            pltpu.VMEM((S, BPW), jnp.int32),
            pltpu.VMEM((NB, CB), jnp.float32),
            pltpu.VMEM((NB, CB), jnp.float32),
            pltpu.VMEM((BPW,), jnp.float32),
            pltpu.VMEM((BPW,), jnp.float32),
            pltpu.VMEM((2, L), jnp.float32),
            pltpu.SemaphoreType.DMA((NB,)),
            pltpu.SemaphoreType.DMA((NB,)),
        ],
    )
    def sc_sum(x_hbm, p0_hbm, p1_hbm, bias_hbm, out_hbm,
               idx_v, b0_v, b1_v, a0_v, a1_v, bv_v, sem0, sem1):
        wid = lax.axis_index("s") * NC + lax.axis_index("c")
        base = wid * BPW
        pltpu.sync_copy(x_hbm.at[:, pl.ds(base, BPW)], idx_v)
        pltpu.sync_copy(bias_hbm, bv_v)

        zero = jnp.zeros((L,), jnp.float32)

        def zbody(i, carry):
            a0_v[pl.ds(i * L, L)] = zero
            a1_v[pl.ds(i * L, L)] = zero
            return carry
        lax.fori_loop(0, BPW // L, zbody, 0)

        T = NCHUNK * S

        def islice(t):
            c = t // S
            s = t - c * S
            return idx_v.at[s, pl.ds(c * CB, CB)]

        def start(t):
            slot = lax.rem(t, NB)
            isl = islice(t)
            pltpu.async_copy(p0_hbm.at[isl], b0_v.at[slot], sem0.at[slot])
            pltpu.async_copy(p1_hbm.at[isl], b1_v.at[slot], sem1.at[slot])

        def prime(t, carry):
            start(t)
            return carry
        lax.fori_loop(0, NB, prime, 0)

        def step(t, carry):
            slot = lax.rem(t, NB)
            isl = islice(t)
            rowbase = (t // S) * CB
            pltpu.make_async_copy(p0_hbm.at[isl], b0_v.at[slot],
                                  sem0.at[slot]).wait()
            for r in range(CB // L):
                plsc.addupdate(a0_v.at[pl.ds(rowbase + r * L, L)],
                               b0_v[slot, pl.ds(r * L, L)])
            pltpu.make_async_copy(p1_hbm.at[isl], b1_v.at[slot],
                                  sem1.at[slot]).wait()
            for r in range(CB // L):
                plsc.addupdate(a1_v.at[pl.ds(rowbase + r * L, L)],
                               b1_v[slot, pl.ds(r * L, L)])

            @pl.when(t + NB < T)
            def _():
                start(t + NB)
            return carry
        lax.fori_loop(0, T, step, 0)

        def finish(i, carry):
            sl = pl.ds(i * L, L)
            z0 = a0_v[sl] + bv_v[0]
            z1 = a1_v[sl] + bv_v[1]
            s0 = 1.0 / (1.0 + jnp.exp(-z0))
            s1 = 1.0 / (1.0 + jnp.exp(-z1))
            m = jnp.maximum(s0, s1)
            u = 1.0 + jnp.exp(jnp.minimum(s0, s1) - m)
            y = (u - 1.0) / (u + 1.0)
            y2 = y * y
            ln_u = 2.0 * y * (1.0 + y2 * (
                1.0 / 3.0 + y2 * (0.2 + y2 * (1.0 / 7.0 + y2 * (1.0 / 9.0)))))
            lse = m + ln_u
            a0_v[sl] = s0 - lse
            a1_v[sl] = s1 - lse
            return carry
        lax.fori_loop(0, BPW // L, finish, 0)

        pltpu.sync_copy(a0_v, out_hbm.at[0, pl.ds(base, BPW)])
        pltpu.sync_copy(a1_v, out_hbm.at[1, pl.ds(base, BPW)])

    return sc_sum


def kernel(x, embed_table, W, b):
    S, B = x.shape
    V, D = embed_table.shape
    O = W.shape[0]
    Wp8 = jnp.zeros((8, D), jnp.float32).at[:O, :].set(W)
    p0, p1 = _project_table(Wp8, embed_table.T)
    bvec = jnp.broadcast_to(b.reshape(O, 1), (O, L)).astype(jnp.float32)
    out2 = _make_sc_sum(S, B)(x.astype(jnp.int32), p0, p1, bvec)
    return out2.T


# speedup vs baseline: 1.0063x; 1.0063x over previous
import functools

import jax
import jax.numpy as jnp
from jax import lax
from jax.experimental import pallas as pl
from jax.experimental.pallas import tpu as pltpu
from jax.experimental.pallas import tpu_sc as plsc

NC = 2
NS = 16
L = 16
CB = 128


def _project_body(w_ref, e_ref, o0_ref, o1_ref):
    r = lax.dot_general(
        w_ref[...], e_ref[...], (((1,), (0,)), ((), ())),
        preferred_element_type=jnp.float32,
        precision=lax.Precision.HIGHEST)
    o0_ref[...] = r[0]
    o1_ref[...] = r[1]


def _project_table(Wp8, et):
    V = et.shape[1]
    C = 50176
    grid = pl.cdiv(V, C)
    return pl.pallas_call(
        _project_body,
        grid=(grid,),
        in_specs=[
            pl.BlockSpec((8, et.shape[0]), lambda i: (0, 0)),
            pl.BlockSpec((et.shape[0], C), lambda i: (0, i)),
        ],
        out_specs=[
            pl.BlockSpec((C,), lambda i: (i,)),
            pl.BlockSpec((C,), lambda i: (i,)),
        ],
        out_shape=[
            jax.ShapeDtypeStruct((V,), jnp.float32),
            jax.ShapeDtypeStruct((V,), jnp.float32),
        ],
    )(Wp8, et)


def _epilogue_body(a_ref, bias_ref, o_ref):
    z = a_ref[...] + bias_ref[...]
    s = jax.nn.sigmoid(z)
    m = jnp.max(s, axis=0, keepdims=True)
    lse = m + jnp.log(jnp.sum(jnp.exp(s - m), axis=0, keepdims=True))
    o_ref[...] = s - lse


def _epilogue(acc2, bias_col):
    B = acc2.shape[1]
    BLK = 4096
    return pl.pallas_call(
        _epilogue_body,
        grid=(B // BLK,),
        in_specs=[
            pl.BlockSpec((2, BLK), lambda i: (0, i)),
            pl.BlockSpec((2, 1), lambda i: (0, 0)),
        ],
        out_specs=pl.BlockSpec((2, BLK), lambda i: (0, i)),
        out_shape=jax.ShapeDtypeStruct((2, B), jnp.float32),
    )(acc2, bias_col)


def _make_sc_sum(S, B):
    NW = NC * NS
    BPW = B // NW
    NCHUNK = BPW // CB
    mesh = plsc.VectorSubcoreMesh(
        core_axis_name="c", subcore_axis_name="s",
        num_cores=NC, num_subcores=NS)

    NB = 8

    @functools.partial(
        pl.kernel,
        out_type=jax.ShapeDtypeStruct((2, B), jnp.float32),
        mesh=mesh,
        compiler_params=pltpu.CompilerParams(use_tc_tiling_on_sc=False),
        scratch_types=[
            pltpu.VMEM((S, BPW), jnp.int32),
            pltpu.VMEM((NB, CB), jnp.float32),
            pltpu.VMEM((NB, CB), jnp.float32),
            pltpu.VMEM((BPW,), jnp.float32),
            pltpu.VMEM((BPW,), jnp.float32),
            pltpu.VMEM((2, L), jnp.float32),
            pltpu.SemaphoreType.DMA((NB,)),
            pltpu.SemaphoreType.DMA((NB,)),
        ],
    )
    def sc_sum(x_hbm, p0_hbm, p1_hbm, bias_hbm, out_hbm,
               idx_v, b0_v, b1_v, a0_v, a1_v, bv_v, sem0, sem1):
        wid = lax.axis_index("s") * NC + lax.axis_index("c")
        base = wid * BPW
        pltpu.sync_copy(x_hbm.at[:, pl.ds(base, BPW)], idx_v)
        pltpu.sync_copy(bias_hbm, bv_v)

        T = NCHUNK * S

        def islice(t):
            c = t // S
            s = t - c * S
            return idx_v.at[s, pl.ds(c * CB, CB)]

        def start(t):
            slot = lax.rem(t, NB)
            isl = islice(t)
            pltpu.async_copy(p0_hbm.at[isl], b0_v.at[slot], sem0.at[slot])
            pltpu.async_copy(p1_hbm.at[isl], b1_v.at[slot], sem1.at[slot])

        def prime(t, carry):
            start(t)
            return carry
        lax.fori_loop(0, NB, prime, 0)

        NR = CB // L
        zero16 = jnp.zeros((L,), jnp.float32)

        def chunk_body(c, carry):
            t0 = c * S

            def sbody(s, accs):
                a0s, a1s = accs
                t = t0 + s
                slot = lax.rem(t, NB)
                isl = islice(t)
                pltpu.make_async_copy(p0_hbm.at[isl], b0_v.at[slot],
                                      sem0.at[slot]).wait()
                a0s = tuple(a0s[r] + b0_v[slot, pl.ds(r * L, L)]
                            for r in range(NR))
                pltpu.make_async_copy(p1_hbm.at[isl], b1_v.at[slot],
                                      sem1.at[slot]).wait()
                a1s = tuple(a1s[r] + b1_v[slot, pl.ds(r * L, L)]
                            for r in range(NR))

                @pl.when(t + NB < T)
                def _():
                    start(t + NB)
                return (a0s, a1s)

            a0s, a1s = lax.fori_loop(
                0, S, sbody, ((zero16,) * NR, (zero16,) * NR))
            for r in range(NR):
                a0_v[pl.ds(c * CB + r * L, L)] = a0s[r]
                a1_v[pl.ds(c * CB + r * L, L)] = a1s[r]
            return carry
        lax.fori_loop(0, NCHUNK, chunk_body, 0)

        def finish(i, carry):
            sl = pl.ds(i * L, L)
            z0 = a0_v[sl] + bv_v[0]
            z1 = a1_v[sl] + bv_v[1]
            s0 = 1.0 / (1.0 + jnp.exp(-z0))
            s1 = 1.0 / (1.0 + jnp.exp(-z1))
            m = jnp.maximum(s0, s1)
            u = 1.0 + jnp.exp(jnp.minimum(s0, s1) - m)
            y = (u - 1.0) / (u + 1.0)
            y2 = y * y
            ln_u = 2.0 * y * (1.0 + y2 * (
                1.0 / 3.0 + y2 * (0.2 + y2 * (1.0 / 7.0 + y2 * (1.0 / 9.0)))))
            lse = m + ln_u
            a0_v[sl] = s0 - lse
            a1_v[sl] = s1 - lse
            return carry
        lax.fori_loop(0, BPW // L, finish, 0)

        pltpu.sync_copy(a0_v, out_hbm.at[0, pl.ds(base, BPW)])
        pltpu.sync_copy(a1_v, out_hbm.at[1, pl.ds(base, BPW)])

    return sc_sum


def kernel(x, embed_table, W, b):
    S, B = x.shape
    V, D = embed_table.shape
    O = W.shape[0]
    Wp8 = jnp.zeros((8, D), jnp.float32).at[:O, :].set(W)
    p0, p1 = _project_table(Wp8, embed_table.T)
    bvec = jnp.broadcast_to(b.reshape(O, 1), (O, L)).astype(jnp.float32)
    out2 = _make_sc_sum(S, B)(x.astype(jnp.int32), p0, p1, bvec)
    return out2.T
